# Initial kernel scaffold; baseline (speedup 1.0000x reference)
import functools

import jax
import jax.numpy as jnp
from jax import lax
from jax.experimental import pallas as pl
from jax.experimental.pallas import tpu as pltpu
from jax.experimental.pallas import tpu_sc as plsc

N_NODES = 10000
IN_DIM = 128
HID = 64
NC = 2
NS = 16
NW = NC * NS
CHUNK = 128
N_PAD = 10240
ROWS_PER_TILE = N_PAD // NS


def _sc_scatter_body(n_chunks, gather_table, src_hbm, dst_hbm, table_hbm,
                     zeros_hbm, ones_hbm, out_hbm, acc_sh, idx_s, idx_d,
                     rows, sem):
    c = lax.axis_index("c")
    s = lax.axis_index("s")
    wid = s * NC + c

    pltpu.sync_copy(zeros_hbm, rows)
    for k in range(ROWS_PER_TILE // CHUNK):
        pltpu.sync_copy(
            rows, acc_sh.at[pl.ds(s * ROWS_PER_TILE + k * CHUNK, CHUNK)])
    plsc.subcore_barrier()

    pltpu.sync_copy(dst_hbm.at[pl.ds(wid * n_chunks, n_chunks)], idx_d)
    if gather_table:
        pltpu.sync_copy(src_hbm.at[pl.ds(wid * n_chunks, n_chunks)], idx_s)
    else:
        pltpu.sync_copy(ones_hbm, rows)

    def body(j, carry):
        if gather_table:
            pltpu.async_copy(table_hbm.at[idx_s.at[j]], rows, sem).wait()
        pltpu.sync_copy(rows, acc_sh.at[idx_d.at[j]], add=True)
        return carry

    lax.fori_loop(0, n_chunks, body, 0)
    plsc.subcore_barrier()

    for k in range(ROWS_PER_TILE // CHUNK):
        sl = pl.ds(s * ROWS_PER_TILE + k * CHUNK, CHUNK)
        pltpu.sync_copy(acc_sh.at[sl], rows)
        pltpu.sync_copy(rows, out_hbm.at[c, sl])


def _make_sc_call(n_chunks, width, gather_table):
    mesh = plsc.VectorSubcoreMesh(core_axis_name="c", subcore_axis_name="s")
    return pl.kernel(
        functools.partial(_sc_scatter_body, n_chunks, gather_table),
        out_type=jax.ShapeDtypeStruct((NC, N_PAD, width), jnp.float32),
        mesh=mesh,
        scratch_types=[
            pltpu.VMEM_SHARED((N_PAD, width), jnp.float32),
            pltpu.VMEM((n_chunks, CHUNK), jnp.int32),
            pltpu.VMEM((n_chunks, CHUNK), jnp.int32),
            pltpu.VMEM((CHUNK, width), jnp.float32),
            pltpu.SemaphoreType.DMA,
        ],
    )


def _dense1_body(degp_ref, x_ref, w1_ref, g_ref, dis_ref):
    deg = degp_ref[0, :, 0:1] + degp_ref[1, :, 0:1] + 1.0
    dis = lax.rsqrt(deg)[:N_NODES, :]
    h = jnp.dot(x_ref[...], w1_ref[...], preferred_element_type=jnp.float32)
    g_ref[...] = h * dis
    dis_ref[...] = dis


def _dense2_body(acc_ref, dis_ref, g1_ref, b1_ref, w2_ref, g2_ref):
    dis = dis_ref[...]
    s = acc_ref[0, :N_NODES, :] + acc_ref[1, :N_NODES, :] + g1_ref[...]
    r = jnp.maximum(s * dis + b1_ref[...], 0.0)
    h2 = jnp.dot(r, w2_ref[...], preferred_element_type=jnp.float32)
    g2_ref[...] = h2 * dis


def _dense3_body(acc_ref, dis_ref, g2_ref, b2_ref, wc_ref, bc_ref, out_ref):
    s = acc_ref[0, :N_NODES, :] + acc_ref[1, :N_NODES, :] + g2_ref[...]
    h = s * dis_ref[...] + b2_ref[...]
    out_ref[...] = (
        jnp.dot(h, wc_ref[...], preferred_element_type=jnp.float32)
        + bc_ref[...])


def kernel(x, edge_index, W1, b1, W2, b2, Wc, bc):
    src = edge_index[0].astype(jnp.int32)
    dst = edge_index[1].astype(jnp.int32)
    n_edges = src.shape[0]
    n_chunks = -(-n_edges // (NW * CHUNK))
    e_pad = NW * n_chunks * CHUNK
    src_p = jnp.concatenate(
        [src, jnp.zeros((e_pad - n_edges,), jnp.int32)]).reshape(-1, CHUNK)
    dst_p = jnp.concatenate(
        [dst, jnp.full((e_pad - n_edges,), N_NODES, jnp.int32)]
    ).reshape(-1, CHUNK)

    zeros16 = jnp.zeros((CHUNK, 16), jnp.float32)
    ones16 = jnp.ones((CHUNK, 16), jnp.float32)
    zeros64 = jnp.zeros((CHUNK, HID), jnp.float32)

    deg_call = _make_sc_call(n_chunks, 16, gather_table=False)
    prop_call = _make_sc_call(n_chunks, HID, gather_table=True)

    degp = deg_call(src_p, dst_p, zeros16, zeros16, ones16)

    g1, dis = pl.pallas_call(
        _dense1_body,
        out_shape=(jax.ShapeDtypeStruct((N_NODES, HID), jnp.float32),
                   jax.ShapeDtypeStruct((N_NODES, 1), jnp.float32)),
    )(degp, x, W1)

    acc1 = prop_call(src_p, dst_p, g1, zeros64, zeros64)

    g2 = pl.pallas_call(
        _dense2_body,
        out_shape=jax.ShapeDtypeStruct((N_NODES, HID), jnp.float32),
    )(acc1, dis, g1, b1.reshape(1, HID), W2)

    acc2 = prop_call(src_p, dst_p, g2, zeros64, zeros64)

    out = pl.pallas_call(
        _dense3_body,
        out_shape=jax.ShapeDtypeStruct((N_NODES, 1), jnp.float32),
    )(acc2, dis, g2, b2.reshape(1, HID), Wc, bc.reshape(1, 1))
    return out


# speedup vs baseline: 13.6982x; 13.6982x over previous
import functools

import jax
import jax.numpy as jnp
from jax import lax
from jax.experimental import pallas as pl
from jax.experimental.pallas import tpu as pltpu
from jax.experimental.pallas import tpu_sc as plsc

N_NODES = 10000
IN_DIM = 128
HID = 64
NC = 2
NS = 16
NW = NC * NS
CHUNK = 128
N_PAD = 10240
ROWS_PER_TILE = N_PAD // NS


def _sc_scatter_body(n_chunks, gather_table, src_hbm, dst_hbm, table_hbm,
                     zeros_hbm, ones_hbm, out_hbm, acc_sh, idx_s, idx_d,
                     rows, sem):
    c = lax.axis_index("c")
    s = lax.axis_index("s")
    wid = s * NC + c

    pltpu.sync_copy(zeros_hbm, rows)
    for k in range(ROWS_PER_TILE // CHUNK):
        pltpu.sync_copy(
            rows, acc_sh.at[pl.ds(s * ROWS_PER_TILE + k * CHUNK, CHUNK)])
    plsc.subcore_barrier()

    pltpu.sync_copy(dst_hbm.at[pl.ds(wid * n_chunks, n_chunks)], idx_d)
    if gather_table:
        pltpu.sync_copy(src_hbm.at[pl.ds(wid * n_chunks, n_chunks)], idx_s)
    else:
        pltpu.sync_copy(ones_hbm, rows)

    def body(j, carry):
        if gather_table:
            pltpu.async_copy(table_hbm.at[idx_s.at[j]], rows, sem).wait()
        pltpu.sync_copy(rows, acc_sh.at[idx_d.at[j]], add=True)
        return carry

    lax.fori_loop(0, n_chunks, body, 0)
    plsc.subcore_barrier()

    for k in range(ROWS_PER_TILE // CHUNK):
        sl = pl.ds(s * ROWS_PER_TILE + k * CHUNK, CHUNK)
        pltpu.sync_copy(acc_sh.at[sl], rows)
        pltpu.sync_copy(rows, out_hbm.at[c, sl])


def _make_sc_call(n_chunks, width, gather_table):
    mesh = plsc.VectorSubcoreMesh(core_axis_name="c", subcore_axis_name="s")
    return pl.kernel(
        functools.partial(_sc_scatter_body, n_chunks, gather_table),
        out_type=jax.ShapeDtypeStruct((NC, N_PAD, width), jnp.float32),
        mesh=mesh,
        scratch_types=[
            pltpu.VMEM_SHARED((N_PAD, width), jnp.float32),
            pltpu.VMEM((n_chunks, CHUNK), jnp.int32),
            pltpu.VMEM((n_chunks, CHUNK), jnp.int32),
            pltpu.VMEM((CHUNK, width), jnp.float32),
            pltpu.SemaphoreType.DMA,
        ],
        compiler_params=pltpu.CompilerParams(use_tc_tiling_on_sc=False),
    )


def _dense1_body(degp_ref, x_ref, w1_ref, g_ref, dis_ref):
    deg = degp_ref[0, :, 0:1] + degp_ref[1, :, 0:1] + 1.0
    dis = lax.rsqrt(deg)[:N_NODES, :]
    h = jnp.dot(x_ref[...], w1_ref[...], preferred_element_type=jnp.float32)
    g_ref[...] = h * dis
    dis_ref[...] = dis


def _dense2_body(acc_ref, dis_ref, g1_ref, b1_ref, w2_ref, g2_ref):
    dis = dis_ref[...]
    s = acc_ref[0, :N_NODES, :] + acc_ref[1, :N_NODES, :] + g1_ref[...]
    r = jnp.maximum(s * dis + b1_ref[...], 0.0)
    h2 = jnp.dot(r, w2_ref[...], preferred_element_type=jnp.float32)
    g2_ref[...] = h2 * dis


def _dense3_body(acc_ref, dis_ref, g2_ref, b2_ref, wc_ref, bc_ref, out_ref):
    s = acc_ref[0, :N_NODES, :] + acc_ref[1, :N_NODES, :] + g2_ref[...]
    h = s * dis_ref[...] + b2_ref[...]
    out_ref[...] = (
        jnp.dot(h, wc_ref[...], preferred_element_type=jnp.float32)
        + bc_ref[...])


def kernel(x, edge_index, W1, b1, W2, b2, Wc, bc):
    src = edge_index[0].astype(jnp.int32)
    dst = edge_index[1].astype(jnp.int32)
    n_edges = src.shape[0]
    n_chunks = -(-n_edges // (NW * CHUNK))
    n_chunks = -(-n_chunks // 8) * 8
    e_pad = NW * n_chunks * CHUNK
    src_p = jnp.concatenate(
        [src, jnp.zeros((e_pad - n_edges,), jnp.int32)]).reshape(-1, CHUNK)
    dst_p = jnp.concatenate(
        [dst, jnp.full((e_pad - n_edges,), N_NODES, jnp.int32)]
    ).reshape(-1, CHUNK)

    zeros16 = jnp.zeros((CHUNK, 16), jnp.float32)
    ones16 = jnp.ones((CHUNK, 16), jnp.float32)
    zeros64 = jnp.zeros((CHUNK, HID), jnp.float32)

    deg_call = _make_sc_call(n_chunks, 16, gather_table=False)
    prop_call = _make_sc_call(n_chunks, HID, gather_table=True)

    degp = deg_call(src_p, dst_p, zeros16, zeros16, ones16)

    g1, dis = pl.pallas_call(
        _dense1_body,
        out_shape=(jax.ShapeDtypeStruct((N_NODES, HID), jnp.float32),
                   jax.ShapeDtypeStruct((N_NODES, 1), jnp.float32)),
    )(degp, x, W1)

    acc1 = prop_call(src_p, dst_p, g1, zeros64, zeros64)

    g2 = pl.pallas_call(
        _dense2_body,
        out_shape=jax.ShapeDtypeStruct((N_NODES, HID), jnp.float32),
    )(acc1, dis, g1, b1.reshape(1, HID), W2)

    acc2 = prop_call(src_p, dst_p, g2, zeros64, zeros64)

    out = pl.pallas_call(
        _dense3_body,
        out_shape=jax.ShapeDtypeStruct((N_NODES, 1), jnp.float32),
    )(acc2, dis, g2, b2.reshape(1, HID), Wc, bc.reshape(1, 1))
    return out
